# speedup vs baseline: 1.2358x; 1.2358x over previous
import jax
import jax.numpy as jnp
from jax import lax
from jax.experimental import pallas as pl
from jax.experimental.pallas import tpu as pltpu
from jax.experimental.pallas import tpu_sc as plsc

NC, NS = 2, 16
NW = NC * NS
B = 16384
D = 128
CHUNK = 128
CPW = B // (NW * CHUNK)


def _body(labi, subi, labt, subt, out, idxL, idxS, bufL, bufS, semL, semS):
    wid = lax.axis_index("s") * NC + lax.axis_index("c")
    row0 = wid * CPW
    pltpu.sync_copy(labi.at[pl.ds(row0, CPW)], idxL)
    pltpu.sync_copy(subi.at[pl.ds(row0, CPW)], idxS)
    for j in range(CPW):
        cl = pltpu.async_copy(labt.at[idxL.at[j]], bufL, semL)
        cs = pltpu.async_copy(subt.at[idxS.at[j]], bufS, semS)
        cl.wait()
        cs.wait()
        r = (row0 + j) * CHUNK
        pltpu.sync_copy(bufL, out.at[pl.ds(r, CHUNK), 0])
        pltpu.sync_copy(bufS, out.at[pl.ds(r, CHUNK), 1])


def kernel(lab_ids, subject_ids, lab_table, subject_table):
    labi = lab_ids.astype(jnp.int32).reshape(B // CHUNK, CHUNK)
    subi = subject_ids.astype(jnp.int32).reshape(B // CHUNK, CHUNK)
    mesh = plsc.VectorSubcoreMesh(core_axis_name="c", subcore_axis_name="s")
    f = pl.kernel(
        _body,
        mesh=mesh,
        out_type=jax.ShapeDtypeStruct((B, 2, D), jnp.float32),
        scratch_types=[
            pltpu.VMEM((CPW, CHUNK), jnp.int32),
            pltpu.VMEM((CPW, CHUNK), jnp.int32),
            pltpu.VMEM((CHUNK, D), jnp.float32),
            pltpu.VMEM((CHUNK, D), jnp.float32),
            pltpu.SemaphoreType.DMA,
            pltpu.SemaphoreType.DMA,
        ],
    )
    out = f(labi, subi, lab_table, subject_table)
    return out.reshape(B, 2 * D)
